# baseline (speedup 1.0000x reference)
import functools

import jax
import jax.numpy as jnp
from jax import lax
from jax.experimental import pallas as pl
from jax.experimental.pallas import tpu as pltpu
from jax.experimental.pallas import tpu_sc as plsc

B = 4096
L = 200
D = 64
NC = 2
NS = 16
NW = NC * NS
RPW = B // NW
LANES = 16
NCHUNK = (L + LANES - 1) // LANES
LP = NCHUNK * LANES
C0 = 104
C1 = L - C0

NEG_INF = float("-inf")


def _body(inputs_hbm, emb_hbm, wk_hbm, wot_hbm, out_hbm,
          idxb, xb0, xb1, lbuf, pbuf, wk, wot, obuf, sem0, sem1):
    cid = lax.axis_index("c")
    sid = lax.axis_index("s")
    wid = sid * NC + cid
    base = wid * RPW

    pltpu.sync_copy(wk_hbm, wk)
    pltpu.sync_copy(wot_hbm, wot)
    pltpu.sync_copy(inputs_hbm.at[pl.ds(base, RPW), :], idxb.at[:, pl.ds(0, L)])

    lane = lax.iota(jnp.int32, 16)
    wkc = [wk[pl.ds(i * 16, 16)] for i in range(4)]

    def start_gather(r, xb, sem):
        pltpu.async_copy(emb_hbm.at[idxb.at[r, pl.ds(0, C0)]],
                         xb.at[pl.ds(0, C0), :], sem)
        pltpu.async_copy(emb_hbm.at[idxb.at[r, pl.ds(C0, C1)]],
                         xb.at[pl.ds(C0, C1), :], sem)

    def wait_gather(xb, sem):
        pltpu.make_async_copy(emb_hbm.at[pl.ds(0, L)],
                              xb.at[pl.ds(0, L), :], sem).wait()

    def compute_row(r, xb):
        def chunk_scores(k, mx):
            rows = k * LANES + lane
            acc = jnp.zeros((16,), jnp.float32)
            for d in range(D):
                v = plsc.load_gather(
                    xb, [rows, jnp.full((16,), d, jnp.int32)])
                acc = acc + v * wkc[d // 16][d % 16]
            idxv = idxb[r, pl.ds(k * LANES, 16)]
            valid = (idxv != 0) & (k * LANES + lane < L)
            e2 = jnp.exp(acc + acc)
            t = 1.0 - 2.0 / (e2 + 1.0)
            z = jnp.where(valid, t, NEG_INF)
            lbuf[pl.ds(k * LANES, 16)] = z
            return jnp.maximum(mx, z)
        mxv = lax.fori_loop(0, NCHUNK, chunk_scores,
                            jnp.full((16,), NEG_INF, jnp.float32))
        m = jnp.max(mxv)

        def chunk_p(k, sv):
            z = lbuf[pl.ds(k * LANES, 16)]
            p = jnp.where(z > NEG_INF, jnp.exp(z - m), 0.0)
            pbuf[pl.ds(k * LANES, 16)] = p
            return sv + p
        svec = lax.fori_loop(0, NCHUNK, chunk_p, jnp.zeros((16,), jnp.float32))
        s = jnp.sum(svec)

        def pool_step(i, accs):
            a0, a1, a2, a3 = accs
            for j in range(4):
                l = i * 4 + j
                p = plsc.load_gather(pbuf, [jnp.full((16,), l, jnp.int32)])
                a0 = a0 + p * xb[l, pl.ds(0, 16)]
                a1 = a1 + p * xb[l, pl.ds(16, 16)]
                a2 = a2 + p * xb[l, pl.ds(32, 16)]
                a3 = a3 + p * xb[l, pl.ds(48, 16)]
            return (a0, a1, a2, a3)
        zero = jnp.zeros((16,), jnp.float32)
        a0, a1, a2, a3 = lax.fori_loop(0, L // 4, pool_step,
                                       (zero, zero, zero, zero))

        t0 = (a0 * wot[0, pl.ds(0, 16)] + a1 * wot[0, pl.ds(16, 16)]
              + a2 * wot[0, pl.ds(32, 16)] + a3 * wot[0, pl.ds(48, 16)])
        t1 = (a0 * wot[1, pl.ds(0, 16)] + a1 * wot[1, pl.ds(16, 16)]
              + a2 * wot[1, pl.ds(32, 16)] + a3 * wot[1, pl.ds(48, 16)])
        ov = jnp.where(lane == 0, jnp.sum(t0), jnp.where(lane == 1, jnp.sum(t1), 0.0))
        ov = ov / jnp.full((16,), s, jnp.float32)
        obuf[r, pl.ds(0, 16)] = ov

    start_gather(0, xb0, sem0)

    def row_pair(g, _):
        r0 = 2 * g
        r1 = 2 * g + 1
        start_gather(r1, xb1, sem1)
        wait_gather(xb0, sem0)
        compute_row(r0, xb0)
        start_gather(jnp.minimum(r0 + 2, RPW - 1), xb0, sem0)
        wait_gather(xb1, sem1)
        compute_row(r1, xb1)
        return 0
    lax.fori_loop(0, RPW // 2, row_pair, 0)
    wait_gather(xb0, sem0)

    pltpu.sync_copy(obuf.at[:, pl.ds(0, 2)], out_hbm.at[pl.ds(base, RPW), :])


@jax.jit
def kernel(inputs, embedding, Wk, Wo, bo):
    wk = Wk[:, 0]
    wot = Wo.T
    run = pl.kernel(
        _body,
        out_type=jax.ShapeDtypeStruct((B, 2), jnp.float32),
        mesh=plsc.VectorSubcoreMesh(core_axis_name="c", subcore_axis_name="s",
                                    num_cores=NC, num_subcores=NS),
        compiler_params=pltpu.CompilerParams(use_tc_tiling_on_sc=False,
                                             needs_layout_passes=False),
        scratch_types=[
            pltpu.VMEM((RPW, LP), jnp.int32),
            pltpu.VMEM((LP, D), jnp.float32),
            pltpu.VMEM((LP, D), jnp.float32),
            pltpu.VMEM((LP,), jnp.float32),
            pltpu.VMEM((LP,), jnp.float32),
            pltpu.VMEM((D,), jnp.float32),
            pltpu.VMEM((2, D), jnp.float32),
            pltpu.VMEM((RPW, 16), jnp.float32),
            pltpu.SemaphoreType.DMA,
            pltpu.SemaphoreType.DMA,
        ],
    )
    out = run(inputs, embedding, wk, wot)
    return out + bo[None, :]
